# baseline (speedup 1.0000x reference)
import functools

import jax
import jax.numpy as jnp
from jax import lax
from jax.experimental import pallas as pl
from jax.experimental.pallas import tpu as pltpu

B, SRC_LEN, TRG_LEN = 32, 50, 8
VOCAB, EMB, HID = 100000, 256, 512
TOPK = 10
VBLK = 2048
NV = (VOCAB + VBLK - 1) // VBLK
NEG = -1e30
BIGI = 2**31 - 1

_interpret = False


def _lstm_math(g, c):
    i = jax.nn.sigmoid(g[:, :HID])
    f = jax.nn.sigmoid(g[:, HID:2 * HID])
    gg = jnp.tanh(g[:, 2 * HID:3 * HID])
    o = jax.nn.sigmoid(g[:, 3 * HID:])
    c = f * c + i * gg
    h = o * jnp.tanh(c)
    return h, c


def _enc_kernel(xe_ref, wx_ref, wh_ref, b_ref, outs_ref, h_ref, c_ref):
    wx = wx_ref[...]
    wh = wh_ref[...]
    b = b_ref[...]

    def step(t, hc):
        h, c = hc
        x = xe_ref[t]
        g = (jnp.dot(x, wx, preferred_element_type=jnp.float32)
             + jnp.dot(h, wh, preferred_element_type=jnp.float32) + b)
        h, c = _lstm_math(g, c)
        outs_ref[t] = h
        return (h, c)

    z = jnp.zeros((B, HID), jnp.float32)
    h, c = lax.fori_loop(0, SRC_LEN, step, (z, z))
    h_ref[...] = h
    c_ref[...] = c


def _proj_topk_kernel(h_ref, w_ref, b_ref, out_ref, tv_ref, ti_ref,
                      rv_ref, ri_ref):
    v = pl.program_id(0)

    @pl.when(v == 0)
    def _():
        rv_ref[...] = jnp.full((B, 16), NEG, jnp.float32)
        ri_ref[...] = jnp.full((B, 16), BIGI, jnp.int32)

    x = (jnp.dot(h_ref[...], w_ref[...], preferred_element_type=jnp.float32)
         + b_ref[...])
    out_ref[...] = x[:, None, :]
    liota = lax.broadcasted_iota(jnp.int32, (B, VBLK), 1)
    xm = jnp.where(v * VBLK + liota < VOCAB, x, NEG)
    rv = rv_ref[...]
    ri = ri_ref[...]
    lane16 = lax.broadcasted_iota(jnp.int32, (B, 16), 1)
    ov = jnp.full((B, 16), NEG, jnp.float32)
    oi = jnp.full((B, 16), BIGI, jnp.int32)
    for i in range(TOPK):
        bm = jnp.max(xm, axis=1, keepdims=True)
        bai = jnp.argmax(xm, axis=1)[:, None]
        rm = jnp.max(rv, axis=1, keepdims=True)
        rai = jnp.argmax(rv, axis=1)[:, None]
        use_r = rm >= bm
        ridx = jnp.sum(jnp.where(lane16 == rai, ri, 0), axis=1,
                       keepdims=True)
        m = jnp.where(use_r, rm, bm)
        idx = jnp.where(use_r, ridx, v * VBLK + bai)
        ov = jnp.where(lane16 == i, m, ov)
        oi = jnp.where(lane16 == i, idx, oi)
        xm = jnp.where((~use_r) & (liota == bai), NEG, xm)
        rv = jnp.where(use_r & (lane16 == rai), NEG, rv)
    rv_ref[...] = ov
    ri_ref[...] = oi

    @pl.when(v == NV - 1)
    def _():
        tv_ref[...] = ov
        ti_ref[...] = oi


def _encoder(src_e, Wx_e, Wh_e, b_e):
    return pl.pallas_call(
        _enc_kernel,
        out_shape=[jax.ShapeDtypeStruct((SRC_LEN, B, HID), jnp.float32),
                   jax.ShapeDtypeStruct((B, HID), jnp.float32),
                   jax.ShapeDtypeStruct((B, HID), jnp.float32)],
        interpret=_interpret,
    )(src_e, Wx_e, Wh_e, b_e.reshape(1, 4 * HID))


def _proj_topk(h, W_out, b_out2):
    return pl.pallas_call(
        _proj_topk_kernel,
        grid=(NV,),
        in_specs=[pl.BlockSpec((B, HID), lambda v: (0, 0)),
                  pl.BlockSpec((HID, VBLK), lambda v: (0, v)),
                  pl.BlockSpec((1, VBLK), lambda v: (0, v))],
        out_specs=[pl.BlockSpec((B, 1, VBLK), lambda v: (0, 0, v)),
                   pl.BlockSpec((B, 16), lambda v: (0, 0)),
                   pl.BlockSpec((B, 16), lambda v: (0, 0))],
        out_shape=[jax.ShapeDtypeStruct((B, 1, VOCAB), jnp.float32),
                   jax.ShapeDtypeStruct((B, 16), jnp.float32),
                   jax.ShapeDtypeStruct((B, 16), jnp.int32)],
        scratch_shapes=[pltpu.VMEM((B, 16), jnp.float32),
                        pltpu.VMEM((B, 16), jnp.int32)],
        interpret=_interpret,
    )(h, W_out, b_out2)


def kernel(src, trg, enc_emb, dec_emb, Wx_e, Wh_e, b_e, Wx_d, Wh_d, b_d,
           W_out, b_out):
    src_e = jnp.swapaxes(jnp.take(enc_emb, src, axis=0), 0, 1)
    enc_outs, hidden, cell = _encoder(src_e, Wx_e, Wh_e, b_e)
    encoder_outputs = jnp.swapaxes(enc_outs, 0, 1)
    b_out2 = b_out.reshape(1, VOCAB)
    samp_key = jax.random.key(42)
    outs = [jnp.zeros((B, 1, VOCAB), jnp.float32)]
    inp_tok = trg[:, 0]
    for t in range(1, TRG_LEN):
        emb_t = jnp.take(dec_emb, inp_tok, axis=0)
        scores = jnp.einsum('bsh,bh->bs', encoder_outputs, hidden)
        attn = jax.nn.softmax(scores, axis=-1)
        context = jnp.einsum('bs,bsh->bh', attn, encoder_outputs)
        x_t = jnp.concatenate([emb_t, context], axis=-1)
        gates = x_t @ Wx_d + hidden @ Wh_d + b_d
        i = jax.nn.sigmoid(gates[:, :HID])
        f = jax.nn.sigmoid(gates[:, HID:2 * HID])
        o = jax.nn.sigmoid(gates[:, 3 * HID:])
        g = jnp.tanh(gates[:, 2 * HID:3 * HID])
        cell = f * cell + i * g
        hidden = o * jnp.tanh(cell)
        logits3, tv, ti = _proj_topk(hidden, W_out, b_out2)
        outs.append(logits3)
        noise = jax.random.gumbel(jax.random.fold_in(samp_key, t),
                                  (B, TOPK), jnp.float32)
        samp = jnp.argmax(tv[:, :TOPK] + noise, axis=-1)
        inp_tok = jnp.take_along_axis(ti[:, :TOPK], samp[:, None],
                                      axis=1).squeeze(1)
    return jnp.concatenate(outs, axis=1)


# speedup vs baseline: 1.0977x; 1.0977x over previous
import functools

import jax
import jax.numpy as jnp
from jax import lax
from jax.experimental import pallas as pl
from jax.experimental.pallas import tpu as pltpu

B, SRC_LEN, TRG_LEN = 32, 50, 8
VOCAB, EMB, HID = 100000, 256, 512
TOPK = 10
VBLK = 4096
NV = (VOCAB + VBLK - 1) // VBLK
NEG = -1e30
BIGI = 2**31 - 1

_interpret = False


def _lstm_math(g, c):
    i = jax.nn.sigmoid(g[:, :HID])
    f = jax.nn.sigmoid(g[:, HID:2 * HID])
    gg = jnp.tanh(g[:, 2 * HID:3 * HID])
    o = jax.nn.sigmoid(g[:, 3 * HID:])
    c = f * c + i * gg
    h = o * jnp.tanh(c)
    return h, c


def _enc_kernel(xe_ref, wx_ref, wh_ref, b_ref, outs_ref, h_ref, c_ref):
    wx = wx_ref[...]
    wh = wh_ref[...]
    b = b_ref[...]

    def step(t, hc):
        h, c = hc
        x = xe_ref[t]
        g = (jnp.dot(x, wx, preferred_element_type=jnp.float32)
             + jnp.dot(h, wh, preferred_element_type=jnp.float32) + b)
        h, c = _lstm_math(g, c)
        outs_ref[t] = h
        return (h, c)

    z = jnp.zeros((B, HID), jnp.float32)
    h, c = lax.fori_loop(0, SRC_LEN, step, (z, z))
    h_ref[...] = h
    c_ref[...] = c


def _proj_topk_kernel(h_ref, w_ref, b_ref, out_ref, tv_ref, ti_ref,
                      rv_ref, ri_ref):
    v = pl.program_id(0)

    @pl.when(v == 0)
    def _():
        rv_ref[...] = jnp.full((B, 16), NEG, jnp.float32)
        ri_ref[...] = jnp.full((B, 16), BIGI, jnp.int32)

    x = (jnp.dot(h_ref[...], w_ref[...], preferred_element_type=jnp.float32)
         + b_ref[...])
    out_ref[...] = x[:, None, :]
    gcol = v * VBLK + lax.broadcasted_iota(jnp.int32, (B, VBLK), 1)
    xm = jnp.where(gcol < VOCAB, x, NEG)
    rv = rv_ref[...]
    ri = ri_ref[...]
    lane16 = lax.broadcasted_iota(jnp.int32, (B, 16), 1)
    ov = jnp.full((B, 16), NEG, jnp.float32)
    oi = jnp.full((B, 16), BIGI, jnp.int32)
    for i in range(TOPK):
        m = jnp.maximum(jnp.max(xm, axis=1, keepdims=True),
                        jnp.max(rv, axis=1, keepdims=True))
        e1 = xm == m
        e2 = rv == m
        idx = jnp.minimum(
            jnp.min(jnp.where(e1, gcol, BIGI), axis=1, keepdims=True),
            jnp.min(jnp.where(e2, ri, BIGI), axis=1, keepdims=True))
        ov = jnp.where(lane16 == i, m, ov)
        oi = jnp.where(lane16 == i, idx, oi)
        xm = jnp.where(e1 & (gcol == idx), NEG, xm)
        rv = jnp.where(e2 & (ri == idx), NEG, rv)
    rv_ref[...] = ov
    ri_ref[...] = oi

    @pl.when(v == NV - 1)
    def _():
        tv_ref[...] = ov
        ti_ref[...] = oi


def _encoder(src_e, Wx_e, Wh_e, b_e):
    return pl.pallas_call(
        _enc_kernel,
        out_shape=[jax.ShapeDtypeStruct((SRC_LEN, B, HID), jnp.float32),
                   jax.ShapeDtypeStruct((B, HID), jnp.float32),
                   jax.ShapeDtypeStruct((B, HID), jnp.float32)],
        interpret=_interpret,
    )(src_e, Wx_e, Wh_e, b_e.reshape(1, 4 * HID))


def _proj_topk(h, W_out, b_out2):
    return pl.pallas_call(
        _proj_topk_kernel,
        grid=(NV,),
        in_specs=[pl.BlockSpec((B, HID), lambda v: (0, 0)),
                  pl.BlockSpec((HID, VBLK), lambda v: (0, v)),
                  pl.BlockSpec((1, VBLK), lambda v: (0, v))],
        out_specs=[pl.BlockSpec((B, 1, VBLK), lambda v: (0, 0, v)),
                   pl.BlockSpec((B, 16), lambda v: (0, 0)),
                   pl.BlockSpec((B, 16), lambda v: (0, 0))],
        out_shape=[jax.ShapeDtypeStruct((B, 1, VOCAB), jnp.float32),
                   jax.ShapeDtypeStruct((B, 16), jnp.float32),
                   jax.ShapeDtypeStruct((B, 16), jnp.int32)],
        scratch_shapes=[pltpu.VMEM((B, 16), jnp.float32),
                        pltpu.VMEM((B, 16), jnp.int32)],
        interpret=_interpret,
    )(h, W_out, b_out2)


def kernel(src, trg, enc_emb, dec_emb, Wx_e, Wh_e, b_e, Wx_d, Wh_d, b_d,
           W_out, b_out):
    src_e = jnp.swapaxes(jnp.take(enc_emb, src, axis=0), 0, 1)
    enc_outs, hidden, cell = _encoder(src_e, Wx_e, Wh_e, b_e)
    encoder_outputs = jnp.swapaxes(enc_outs, 0, 1)
    b_out2 = b_out.reshape(1, VOCAB)
    samp_key = jax.random.key(42)
    outs = [jnp.zeros((B, 1, VOCAB), jnp.float32)]
    inp_tok = trg[:, 0]
    for t in range(1, TRG_LEN):
        emb_t = jnp.take(dec_emb, inp_tok, axis=0)
        scores = jnp.einsum('bsh,bh->bs', encoder_outputs, hidden)
        attn = jax.nn.softmax(scores, axis=-1)
        context = jnp.einsum('bs,bsh->bh', attn, encoder_outputs)
        x_t = jnp.concatenate([emb_t, context], axis=-1)
        gates = x_t @ Wx_d + hidden @ Wh_d + b_d
        i = jax.nn.sigmoid(gates[:, :HID])
        f = jax.nn.sigmoid(gates[:, HID:2 * HID])
        o = jax.nn.sigmoid(gates[:, 3 * HID:])
        g = jnp.tanh(gates[:, 2 * HID:3 * HID])
        cell = f * cell + i * g
        hidden = o * jnp.tanh(cell)
        logits3, tv, ti = _proj_topk(hidden, W_out, b_out2)
        outs.append(logits3)
        noise = jax.random.gumbel(jax.random.fold_in(samp_key, t),
                                  (B, TOPK), jnp.float32)
        samp = jnp.argmax(tv[:, :TOPK] + noise, axis=-1)
        inp_tok = jnp.take_along_axis(ti[:, :TOPK], samp[:, None],
                                      axis=1).squeeze(1)
    return jnp.concatenate(outs, axis=1)


# speedup vs baseline: 1.3361x; 1.2172x over previous
import functools

import jax
import jax.numpy as jnp
from jax import lax
from jax.experimental import pallas as pl
from jax.experimental.pallas import tpu as pltpu

B, SRC_LEN, TRG_LEN = 32, 50, 8
VOCAB, EMB, HID = 100000, 256, 512
TOPK = 10
VBLK = 4096
NV = (VOCAB + VBLK - 1) // VBLK
NEG = -1e30
BIGI = 2**31 - 1

_interpret = False


def _lstm_math(g, c):
    i = jax.nn.sigmoid(g[:, :HID])
    f = jax.nn.sigmoid(g[:, HID:2 * HID])
    gg = jnp.tanh(g[:, 2 * HID:3 * HID])
    o = jax.nn.sigmoid(g[:, 3 * HID:])
    c = f * c + i * gg
    h = o * jnp.tanh(c)
    return h, c


def _enc_kernel(xe_ref, wx_ref, wh_ref, b_ref, outs_ref, h_ref, c_ref):
    wx = wx_ref[...]
    wh = wh_ref[...]
    b = b_ref[...]

    def step(t, hc):
        h, c = hc
        x = xe_ref[t]
        g = (jnp.dot(x, wx, preferred_element_type=jnp.float32)
             + jnp.dot(h, wh, preferred_element_type=jnp.float32) + b)
        h, c = _lstm_math(g, c)
        outs_ref[t] = h
        return (h, c)

    z = jnp.zeros((B, HID), jnp.float32)
    h, c = lax.fori_loop(0, SRC_LEN, step, (z, z))
    h_ref[...] = h
    c_ref[...] = c


def _proj_topk_kernel(h_ref, w_ref, b_ref, out_ref, tv_ref, ti_ref,
                      rv_ref, ri_ref):
    v = pl.program_id(0)

    @pl.when(v == 0)
    def _():
        rv_ref[...] = jnp.full((B, 16), NEG, jnp.float32)
        ri_ref[...] = jnp.full((B, 16), BIGI, jnp.int32)

    x = (jnp.dot(h_ref[...], w_ref[...], preferred_element_type=jnp.float32)
         + b_ref[...])
    out_ref[...] = x
    gcol = v * VBLK + lax.broadcasted_iota(jnp.int32, (B, VBLK), 1)
    xm = jnp.where(gcol < VOCAB, x, NEG)
    rv = rv_ref[...]
    ri = ri_ref[...]
    lane16 = lax.broadcasted_iota(jnp.int32, (B, 16), 1)
    ov = jnp.full((B, 16), NEG, jnp.float32)
    oi = jnp.full((B, 16), BIGI, jnp.int32)
    for i in range(TOPK):
        m = jnp.maximum(jnp.max(xm, axis=1, keepdims=True),
                        jnp.max(rv, axis=1, keepdims=True))
        e1 = xm == m
        e2 = rv == m
        idx = jnp.minimum(
            jnp.min(jnp.where(e1, gcol, BIGI), axis=1, keepdims=True),
            jnp.min(jnp.where(e2, ri, BIGI), axis=1, keepdims=True))
        ov = jnp.where(lane16 == i, m, ov)
        oi = jnp.where(lane16 == i, idx, oi)
        xm = jnp.where(e1 & (gcol == idx), NEG, xm)
        rv = jnp.where(e2 & (ri == idx), NEG, rv)
    rv_ref[...] = ov
    ri_ref[...] = oi

    @pl.when(v == NV - 1)
    def _():
        tv_ref[...] = ov
        ti_ref[...] = oi


ABLK = 4096
NA = (VOCAB + ABLK - 1) // ABLK


def _assemble_kernel(*refs):
    out_ref = refs[-1]
    ins = refs[:-1]
    out_ref[:, 0, :] = jnp.zeros((B, ABLK), jnp.float32)
    for i, r in enumerate(ins):
        out_ref[:, i + 1, :] = r[...]


def _assemble(logits_list):
    return pl.pallas_call(
        _assemble_kernel,
        grid=(NA,),
        in_specs=[pl.BlockSpec((B, ABLK), lambda v: (0, v))
                  for _ in logits_list],
        out_specs=pl.BlockSpec((B, TRG_LEN, ABLK), lambda v: (0, 0, v)),
        out_shape=jax.ShapeDtypeStruct((B, TRG_LEN, VOCAB), jnp.float32),
        interpret=_interpret,
    )(*logits_list)


def _encoder(src_e, Wx_e, Wh_e, b_e):
    return pl.pallas_call(
        _enc_kernel,
        out_shape=[jax.ShapeDtypeStruct((SRC_LEN, B, HID), jnp.float32),
                   jax.ShapeDtypeStruct((B, HID), jnp.float32),
                   jax.ShapeDtypeStruct((B, HID), jnp.float32)],
        interpret=_interpret,
    )(src_e, Wx_e, Wh_e, b_e.reshape(1, 4 * HID))


def _proj_topk(h, W_out, b_out2):
    return pl.pallas_call(
        _proj_topk_kernel,
        grid=(NV,),
        in_specs=[pl.BlockSpec((B, HID), lambda v: (0, 0)),
                  pl.BlockSpec((HID, VBLK), lambda v: (0, v)),
                  pl.BlockSpec((1, VBLK), lambda v: (0, v))],
        out_specs=[pl.BlockSpec((B, VBLK), lambda v: (0, v)),
                   pl.BlockSpec((B, 16), lambda v: (0, 0)),
                   pl.BlockSpec((B, 16), lambda v: (0, 0))],
        out_shape=[jax.ShapeDtypeStruct((B, VOCAB), jnp.float32),
                   jax.ShapeDtypeStruct((B, 16), jnp.float32),
                   jax.ShapeDtypeStruct((B, 16), jnp.int32)],
        scratch_shapes=[pltpu.VMEM((B, 16), jnp.float32),
                        pltpu.VMEM((B, 16), jnp.int32)],
        interpret=_interpret,
    )(h, W_out, b_out2)


def kernel(src, trg, enc_emb, dec_emb, Wx_e, Wh_e, b_e, Wx_d, Wh_d, b_d,
           W_out, b_out):
    src_e = jnp.swapaxes(jnp.take(enc_emb, src, axis=0), 0, 1)
    enc_outs, hidden, cell = _encoder(src_e, Wx_e, Wh_e, b_e)
    encoder_outputs = jnp.swapaxes(enc_outs, 0, 1)
    b_out2 = b_out.reshape(1, VOCAB)
    samp_key = jax.random.key(42)
    outs = []
    inp_tok = trg[:, 0]
    for t in range(1, TRG_LEN):
        emb_t = jnp.take(dec_emb, inp_tok, axis=0)
        scores = jnp.einsum('bsh,bh->bs', encoder_outputs, hidden)
        attn = jax.nn.softmax(scores, axis=-1)
        context = jnp.einsum('bs,bsh->bh', attn, encoder_outputs)
        x_t = jnp.concatenate([emb_t, context], axis=-1)
        gates = x_t @ Wx_d + hidden @ Wh_d + b_d
        i = jax.nn.sigmoid(gates[:, :HID])
        f = jax.nn.sigmoid(gates[:, HID:2 * HID])
        o = jax.nn.sigmoid(gates[:, 3 * HID:])
        g = jnp.tanh(gates[:, 2 * HID:3 * HID])
        cell = f * cell + i * g
        hidden = o * jnp.tanh(cell)
        logits, tv, ti = _proj_topk(hidden, W_out, b_out2)
        outs.append(logits)
        noise = jax.random.gumbel(jax.random.fold_in(samp_key, t),
                                  (B, TOPK), jnp.float32)
        samp = jnp.argmax(tv[:, :TOPK] + noise, axis=-1)
        inp_tok = jnp.take_along_axis(ti[:, :TOPK], samp[:, None],
                                      axis=1).squeeze(1)
    return _assemble(outs)
